# baseline (speedup 1.0000x reference)
import jax
import jax.numpy as jnp
from jax import lax
from jax.experimental import pallas as pl
from jax.experimental.pallas import tpu as pltpu
from jax.experimental.pallas import tpu_sc as plsc

VOCAB = 100000
D_MODEL = 128
MAX_LEN = 512
BATCH = 4096
SEQ = 200

NUM_CORES = 2
NUM_SUBCORES = 16
NW = NUM_CORES * NUM_SUBCORES
ROWS_PER_W = BATCH * SEQ // NW
LANES = 16
SLOTS = ((0, 104, 0), (104, 96, 104), (200, 104, 0), (304, 96, 104))
NBUF = 8
NSLOT = ROWS_PER_W // 100
NI = NSLOT // NBUF


def _sine_pe():
    pos = jnp.arange(MAX_LEN, dtype=jnp.float32)[:, None]
    div = jnp.exp(
        jnp.arange(0, D_MODEL, 2, dtype=jnp.float32)
        * (-jnp.log(10000.0) / D_MODEL)
    )
    pe = jnp.zeros((MAX_LEN, D_MODEL), dtype=jnp.float32)
    pe = pe.at[:, 0::2].set(jnp.sin(pos * div))
    pe = pe.at[:, 1::2].set(jnp.cos(pos * div))
    return pe[:SEQ]


def _body(ids, table, pe, out, pe_v, *refs):
    rows = refs[:NBUF]
    idxs = refs[NBUF:2 * NBUF]
    gs = refs[2 * NBUF:3 * NBUF]
    os_ = refs[3 * NBUF:4 * NBUF]
    is_ = refs[4 * NBUF:5 * NBUF]

    c = lax.axis_index("c")
    s = lax.axis_index("s")
    wid = s * NUM_CORES + c
    base = wid * ROWS_PER_W

    pltpu.sync_copy(pe, pe_v)

    def lo_of(q, j):
        return q * 400 + SLOTS[j % 4][0]

    def idx_copy(q, j):
        ln = SLOTS[j % 4][1]
        return pltpu.make_async_copy(
            ids.at[pl.ds(base + lo_of(q, j), ln)], idxs[j], is_[j]
        )

    def gather_copy(q, j):
        return pltpu.make_async_copy(
            table.at[idxs[j]], rows[j], gs[j]
        )

    def out_copy(q, j):
        ln = SLOTS[j % 4][1]
        return pltpu.make_async_copy(
            rows[j], out.at[pl.ds(base + lo_of(q, j), ln)], os_[j]
        )

    def add_pe(j):
        ln, pe_off = SLOTS[j % 4][1], SLOTS[j % 4][2]

        @plsc.parallel_loop(0, ln, unroll=4)
        def row_body(r):
            for cc in range(D_MODEL // LANES):
                plsc.addupdate(
                    rows[j].at[r, pl.ds(cc * LANES, LANES)],
                    pe_v[pe_off + r, pl.ds(cc * LANES, LANES)],
                )

    for j in range(6):
        idx_copy(j // 4, j).start()
    for j in range(4):
        idx_copy(0, j).wait()
        gather_copy(0, j).start()

    def pipe_body(i, carry):
        for j in range(NBUF):
            q = 2 * i + (j // 4)
            j4 = (j + 4) % NBUF
            q4 = q + 1

            gather_copy(q, j).wait()

            def start_next():
                @pl.when(q4 >= 2)
                def _():
                    out_copy(q4 - 2, j4).wait()

                idx_copy(q4, j4).wait()
                gather_copy(q4, j4).start()

            if j < 4:
                start_next()
            else:
                @pl.when(i < NI - 1)
                def _():
                    start_next()

            j6 = (j + 6) % NBUF
            q6 = 2 * i + (j + 6) // 4
            if j < 2:
                idx_copy(q6, j6).start()
            else:
                @pl.when(i < NI - 1)
                def _():
                    idx_copy(q6, j6).start()

            add_pe(j)
            out_copy(q, j).start()
        return carry

    lax.fori_loop(0, NI, pipe_body, 0)
    for j in range(NBUF):
        out_copy(62 + j // 4, j).wait()


def kernel(input_ids, emb_table):
    batch, seq = input_ids.shape
    _, d = emb_table.shape
    pe = _sine_pe()
    row_types = [pltpu.VMEM((SLOTS[j % 4][1], D_MODEL), jnp.float32)
                 for j in range(NBUF)]
    idx_types = [pltpu.VMEM((SLOTS[j % 4][1],), jnp.int32)
                 for j in range(NBUF)]
    fn = pl.kernel(
        _body,
        out_type=jax.ShapeDtypeStruct((batch * seq, d), jnp.float32),
        mesh=plsc.VectorSubcoreMesh(
            core_axis_name="c", subcore_axis_name="s"
        ),
        compiler_params=pltpu.CompilerParams(use_tc_tiling_on_sc=False),
        scratch_types=(
            [pltpu.VMEM((SEQ, D_MODEL), jnp.float32)]
            + row_types
            + idx_types
            + [pltpu.SemaphoreType.DMA] * (3 * NBUF)
        ),
    )
    flat = fn(input_ids.reshape(-1).astype(jnp.int32), emb_table, pe)
    return flat.reshape(batch, seq, d)


# speedup vs baseline: 1.0131x; 1.0131x over previous
import jax
import jax.numpy as jnp
from jax import lax
from jax.experimental import pallas as pl
from jax.experimental.pallas import tpu as pltpu
from jax.experimental.pallas import tpu_sc as plsc

VOCAB = 100000
D_MODEL = 128
MAX_LEN = 512
BATCH = 4096
SEQ = 200

NUM_CORES = 2
NUM_SUBCORES = 16
NW = NUM_CORES * NUM_SUBCORES
ROWS_PER_W = BATCH * SEQ // NW
LANES = 16
SLOTS = ((0, 104, 0), (104, 96, 104), (200, 104, 0), (304, 96, 104))
NBUF = 4
NP = ROWS_PER_W // 400


def _sine_pe():
    pos = jnp.arange(MAX_LEN, dtype=jnp.float32)[:, None]
    div = jnp.exp(
        jnp.arange(0, D_MODEL, 2, dtype=jnp.float32)
        * (-jnp.log(10000.0) / D_MODEL)
    )
    pe = jnp.zeros((MAX_LEN, D_MODEL), dtype=jnp.float32)
    pe = pe.at[:, 0::2].set(jnp.sin(pos * div))
    pe = pe.at[:, 1::2].set(jnp.cos(pos * div))
    return pe[:SEQ]


def _body(ids, table, pe, out, idx_v, pe_v, r0, r1, r2, r3,
          g0, g1, g2, g3, o0, o1, o2, o3):
    c = lax.axis_index("c")
    s = lax.axis_index("s")
    wid = s * NUM_CORES + c
    base = wid * ROWS_PER_W

    rows = (r0, r1, r2, r3)
    gs = (g0, g1, g2, g3)
    os_ = (o0, o1, o2, o3)

    pltpu.sync_copy(pe, pe_v)
    pltpu.sync_copy(ids.at[pl.ds(base, ROWS_PER_W)], idx_v)

    def gather_copy(p, b):
        off, ln, _ = SLOTS[b]
        lo = p * 400 + off
        return pltpu.make_async_copy(
            table.at[idx_v.at[pl.ds(lo, ln)]],
            rows[b].at[pl.ds(0, ln)],
            gs[b],
        )

    def out_copy(p, b):
        off, ln, _ = SLOTS[b]
        lo = p * 400 + off
        return pltpu.make_async_copy(
            rows[b].at[pl.ds(0, ln)],
            out.at[pl.ds(base + lo, ln)],
            os_[b],
        )

    def add_pe(p, b):
        _, ln, pe_off = SLOTS[b]

        @plsc.parallel_loop(0, ln, unroll=4)
        def row_body(r):
            for cc in range(D_MODEL // LANES):
                plsc.addupdate(
                    rows[b].at[r, pl.ds(cc * LANES, LANES)],
                    pe_v[pe_off + r, pl.ds(cc * LANES, LANES)],
                )

    gather_copy(0, 0).start()
    gather_copy(0, 1).start()

    def pipe_body(p, carry):
        for b in range(NBUF):
            nb = (b + 2) % NBUF
            p2 = p if b < 2 else p + 1

            gather_copy(p, b).wait()

            @pl.when(p2 < NP)
            def _():
                @pl.when(p2 >= 1)
                def _():
                    out_copy(p2 - 1, nb).wait()

                gather_copy(p2, nb).start()

            add_pe(p, b)
            out_copy(p, b).start()
        return carry

    lax.fori_loop(0, NP, pipe_body, 0)
    for b in range(NBUF):
        out_copy(NP - 1, b).wait()


def kernel(input_ids, emb_table):
    batch, seq = input_ids.shape
    _, d = emb_table.shape
    pe = _sine_pe()
    fn = pl.kernel(
        _body,
        out_type=jax.ShapeDtypeStruct((batch * seq, d), jnp.float32),
        mesh=plsc.VectorSubcoreMesh(
            core_axis_name="c", subcore_axis_name="s"
        ),
        compiler_params=pltpu.CompilerParams(use_tc_tiling_on_sc=False),
        scratch_types=(
            [pltpu.VMEM((ROWS_PER_W,), jnp.int32)]
            + [pltpu.VMEM((SEQ, D_MODEL), jnp.float32)]
            + [pltpu.VMEM((104, D_MODEL), jnp.float32)] * NBUF
            + [pltpu.SemaphoreType.DMA] * (2 * NBUF)
        ),
    )
    flat = fn(input_ids.reshape(-1).astype(jnp.int32), emb_table, pe)
    return flat.reshape(batch, seq, d)
